# baseline (speedup 1.0000x reference)
import functools

import jax
import jax.numpy as jnp
from jax.experimental import pallas as pl
from jax.experimental.pallas import tpu as pltpu

N_NODES = 10000
D = 128
D_RBF = 16
OUT = 128

NW = 128
C = 1024
NB = (N_NODES + NW - 1) // NW
NSLOTS = 6


SROWS = 88
SCHUNK = 6400


def _s_kernel(recv_ref, out_ref):
    s = pl.program_id(0)

    @pl.when(s == 0)
    def _():
        out_ref[...] = jnp.zeros_like(out_ref)

    r = recv_ref[...]
    t = NW * jax.lax.broadcasted_iota(jnp.int32, (SROWS, 1), 0)
    lt = (r < t).astype(jnp.int32)
    out_ref[...] += jnp.sum(lt, axis=1, keepdims=True)


def _agg_kernel(S_ref, recv_hbm, x_hbm, rbf_hbm, W1_ref, W2_ref,
                out_ref, x_buf, rbf_buf, r_buf, sem_x, sem_rbf, sem_r, *,
                n_edges):
    j = pl.program_id(0)
    a0 = S_ref[j, 0]
    b0 = S_ref[j + 1, 0]
    al0 = (a0 // 128) * 128
    nchunks = (b0 - al0 + C - 1) // C
    base = j * NW
    rows = base + jax.lax.broadcasted_iota(jnp.int32, (NW, 1), 0)

    def copies(start, slot):
        return (
            pltpu.make_async_copy(x_hbm.at[pl.ds(start, C), :],
                                  x_buf.at[slot], sem_x.at[slot]),
            pltpu.make_async_copy(rbf_hbm.at[pl.ds(start, C), :],
                                  rbf_buf.at[slot], sem_rbf.at[slot]),
            pltpu.make_async_copy(recv_hbm.at[:, pl.ds(start, C)],
                                  r_buf.at[slot], sem_r.at[slot]),
        )

    def chunk_start(al, k):
        return jnp.minimum(al + k * C, n_edges - C)

    def issue(start, slot):
        for cp in copies(start, slot):
            cp.start()

    @pl.when(j == 0)
    def _():
        for m in range(NSLOTS - 1):
            @pl.when(m < nchunks)
            def _():
                issue(chunk_start(al0, m), m)

    def body(k, carry):
        acc, cnt = carry
        slot = jax.lax.rem(k, NSLOTS)
        a = al0 + k * C
        start = chunk_start(al0, k)

        @pl.when(k + NSLOTS - 1 < nchunks)
        def _():
            issue(chunk_start(al0, k + NSLOTS - 1),
                  jax.lax.rem(k + NSLOTS - 1, NSLOTS))

        for cp in copies(start, slot):
            cp.wait()
        rw = jnp.dot(rbf_buf[slot].astype(jnp.bfloat16), W1_ref[...],
                     preferred_element_type=jnp.float32)
        xe = (rw * x_buf[slot]).astype(jnp.bfloat16)
        ge = start + jax.lax.broadcasted_iota(jnp.int32, (1, C), 1)
        valid = (ge >= jnp.maximum(a, a0)) & (ge < jnp.minimum(a + C, b0))
        ohb = (r_buf[slot] == rows) & valid
        oh = ohb.astype(jnp.bfloat16)
        acc = acc + jnp.dot(oh, xe, preferred_element_type=jnp.float32)
        cnt = cnt + jnp.sum(ohb.astype(jnp.float32), axis=1, keepdims=True)
        return acc, cnt

    acc0 = jnp.zeros((NW, D), jnp.float32)
    cnt0 = jnp.zeros((NW, 1), jnp.float32)
    acc, cnt = jax.lax.fori_loop(0, nchunks, body, (acc0, cnt0))

    @pl.when(j + 1 < pl.num_programs(0))
    def _():
        a0n = S_ref[j + 1, 0]
        b0n = S_ref[j + 2, 0]
        al0n = (a0n // 128) * 128
        nchunks_n = (b0n - al0n + C - 1) // C
        for m in range(NSLOTS - 1):
            @pl.when(m < nchunks_n)
            def _():
                issue(chunk_start(al0n, m), m)

    node_x = acc / jnp.maximum(cnt, 1.0)
    out_ref[...] = jnp.dot(node_x, W2_ref[...], preferred_element_type=jnp.float32)


@jax.jit
def kernel(rbf, x, receivers, W1, W2):
    E = x.shape[0]
    receivers = receivers.astype(jnp.int32)
    recv2d = receivers.reshape(1, E)
    S = pl.pallas_call(
        _s_kernel,
        grid=(E // SCHUNK,),
        in_specs=[pl.BlockSpec((1, SCHUNK), lambda s: (0, s))],
        out_specs=pl.BlockSpec((SROWS, 1), lambda s: (0, 0)),
        out_shape=jax.ShapeDtypeStruct((SROWS, 1), jnp.int32),
        compiler_params=pltpu.CompilerParams(
            dimension_semantics=("arbitrary",),
        ),
    )(recv2d)

    out = pl.pallas_call(
        functools.partial(_agg_kernel, n_edges=E),
        grid=(NB,),
        in_specs=[
            pl.BlockSpec(memory_space=pltpu.SMEM),
            pl.BlockSpec(memory_space=pltpu.HBM),
            pl.BlockSpec(memory_space=pltpu.HBM),
            pl.BlockSpec(memory_space=pltpu.HBM),
            pl.BlockSpec((D_RBF, D), lambda j: (0, 0)),
            pl.BlockSpec((D, OUT), lambda j: (0, 0)),
        ],
        out_specs=pl.BlockSpec((NW, OUT), lambda j: (j, 0)),
        out_shape=jax.ShapeDtypeStruct((N_NODES, OUT), jnp.float32),
        scratch_shapes=[
            pltpu.VMEM((NSLOTS, C, D), jnp.float32),
            pltpu.VMEM((NSLOTS, C, D_RBF), jnp.float32),
            pltpu.VMEM((NSLOTS, 1, C), jnp.int32),
            pltpu.SemaphoreType.DMA((NSLOTS,)),
            pltpu.SemaphoreType.DMA((NSLOTS,)),
            pltpu.SemaphoreType.DMA((NSLOTS,)),
        ],
        compiler_params=pltpu.CompilerParams(
            dimension_semantics=("arbitrary",),
        ),
    )(S, recv2d, x, rbf, W1.astype(jnp.bfloat16), W2)
    return out


# speedup vs baseline: 1.0496x; 1.0496x over previous
import functools

import jax
import jax.numpy as jnp
from jax.experimental import pallas as pl
from jax.experimental.pallas import tpu as pltpu

N_NODES = 10000
D = 128
D_RBF = 16
OUT = 128

NW = 128
C = 1024
NB = (N_NODES + NW - 1) // NW
NSLOTS = 6


SROWS = 88
SCHUNK = 6400


def _s_kernel(recv_ref, out_ref, *, n_edges):
    t = NW * jax.lax.broadcasted_iota(jnp.int32, (SROWS, 1), 0)

    def body(s, acc):
        off = pl.multiple_of(s * SCHUNK, 128)
        r = recv_ref[pl.ds(off, SCHUNK)].reshape(1, SCHUNK)
        lt = (r < t).astype(jnp.int32)
        return acc + jnp.sum(lt, axis=1, keepdims=True)

    acc0 = jnp.zeros((SROWS, 1), jnp.int32)
    out_ref[...] = jax.lax.fori_loop(0, n_edges // SCHUNK, body, acc0)


def _agg_kernel(S_ref, recv_hbm, x_hbm, rbf_hbm, W1_ref, W2_ref,
                out_ref, x_buf, rbf_buf, r_buf, sem_x, sem_rbf, sem_r, *,
                n_edges):
    j = pl.program_id(0)
    a0 = S_ref[j, 0]
    b0 = S_ref[j + 1, 0]
    al0 = (a0 // 128) * 128
    nchunks = (b0 - al0 + C - 1) // C
    base = j * NW
    rows = base + jax.lax.broadcasted_iota(jnp.int32, (NW, 1), 0)

    def copies(start, slot):
        return (
            pltpu.make_async_copy(x_hbm.at[pl.ds(start, C), :],
                                  x_buf.at[slot], sem_x.at[slot]),
            pltpu.make_async_copy(rbf_hbm.at[pl.ds(start, C), :],
                                  rbf_buf.at[slot], sem_rbf.at[slot]),
            pltpu.make_async_copy(recv_hbm.at[pl.ds(start, C)],
                                  r_buf.at[slot], sem_r.at[slot]),
        )

    def chunk_start(al, k):
        return pl.multiple_of(jnp.minimum(al + k * C, n_edges - C), 128)

    def issue(start, slot):
        for cp in copies(start, slot):
            cp.start()

    @pl.when(j == 0)
    def _():
        for m in range(NSLOTS - 1):
            @pl.when(m < nchunks)
            def _():
                issue(chunk_start(al0, m), m)

    def body(k, carry):
        acc, cnt = carry
        slot = jax.lax.rem(k, NSLOTS)
        a = al0 + k * C
        start = chunk_start(al0, k)

        @pl.when(k + NSLOTS - 1 < nchunks)
        def _():
            issue(chunk_start(al0, k + NSLOTS - 1),
                  jax.lax.rem(k + NSLOTS - 1, NSLOTS))

        for cp in copies(start, slot):
            cp.wait()
        rw = jnp.dot(rbf_buf[slot].astype(jnp.bfloat16), W1_ref[...],
                     preferred_element_type=jnp.float32)
        xe = (rw * x_buf[slot]).astype(jnp.bfloat16)
        ge = start + jax.lax.broadcasted_iota(jnp.int32, (1, C), 1)
        valid = (ge >= jnp.maximum(a, a0)) & (ge < jnp.minimum(a + C, b0))
        ohb = (r_buf[slot].reshape(1, C) == rows) & valid
        oh = ohb.astype(jnp.bfloat16)
        acc = acc + jnp.dot(oh, xe, preferred_element_type=jnp.float32)
        cnt = cnt + jnp.sum(ohb.astype(jnp.float32), axis=1, keepdims=True)
        return acc, cnt

    acc0 = jnp.zeros((NW, D), jnp.float32)
    cnt0 = jnp.zeros((NW, 1), jnp.float32)
    acc, cnt = jax.lax.fori_loop(0, nchunks, body, (acc0, cnt0))

    @pl.when(j + 1 < pl.num_programs(0))
    def _():
        a0n = S_ref[j + 1, 0]
        b0n = S_ref[j + 2, 0]
        al0n = (a0n // 128) * 128
        nchunks_n = (b0n - al0n + C - 1) // C
        for m in range(NSLOTS - 1):
            @pl.when(m < nchunks_n)
            def _():
                issue(chunk_start(al0n, m), m)

    node_x = acc / jnp.maximum(cnt, 1.0)
    out_ref[...] = jnp.dot(node_x, W2_ref[...], preferred_element_type=jnp.float32)


@jax.jit
def kernel(rbf, x, receivers, W1, W2):
    E = x.shape[0]
    receivers = receivers.astype(jnp.int32)
    S = pl.pallas_call(
        functools.partial(_s_kernel, n_edges=E),
        out_shape=jax.ShapeDtypeStruct((SROWS, 1), jnp.int32),
    )(receivers)

    out = pl.pallas_call(
        functools.partial(_agg_kernel, n_edges=E),
        grid=(NB,),
        in_specs=[
            pl.BlockSpec(memory_space=pltpu.SMEM),
            pl.BlockSpec(memory_space=pltpu.HBM),
            pl.BlockSpec(memory_space=pltpu.HBM),
            pl.BlockSpec(memory_space=pltpu.HBM),
            pl.BlockSpec((D_RBF, D), lambda j: (0, 0)),
            pl.BlockSpec((D, OUT), lambda j: (0, 0)),
        ],
        out_specs=pl.BlockSpec((NW, OUT), lambda j: (j, 0)),
        out_shape=jax.ShapeDtypeStruct((N_NODES, OUT), jnp.float32),
        scratch_shapes=[
            pltpu.VMEM((NSLOTS, C, D), jnp.float32),
            pltpu.VMEM((NSLOTS, C, D_RBF), jnp.float32),
            pltpu.VMEM((NSLOTS, C), jnp.int32),
            pltpu.SemaphoreType.DMA((NSLOTS,)),
            pltpu.SemaphoreType.DMA((NSLOTS,)),
            pltpu.SemaphoreType.DMA((NSLOTS,)),
        ],
        compiler_params=pltpu.CompilerParams(
            dimension_semantics=("arbitrary",),
        ),
    )(S, receivers, x, rbf, W1.astype(jnp.bfloat16), W2)
    return out
